# baseline (speedup 1.0000x reference)
import functools

import jax
import jax.numpy as jnp
from jax import lax
from jax.experimental import pallas as pl
from jax.experimental.pallas import tpu as pltpu
from jax.experimental.pallas import tpu_sc as plsc

N = 10000
D_IN = 128
HID = 48
NEG = 0.2
DEPTH = 5

NPAD = 10240
HPAD = 64
NTILES = 32
EP_TILE = 10496
EPAD = NTILES * EP_TILE
K = 128
NCHUNK = EP_TILE // K
ROWS_PER_TILE = NPAD // 16


_LOG2E = 1.4426950408889634
_LN2_HI = 0.693145751953125
_LN2_LO = 1.42860677e-06


def _soft_exp(x):
    x = jnp.maximum(x, -80.0)
    n = (x * _LOG2E - 0.5).astype(jnp.int32)
    nf = n.astype(jnp.float32)
    r = x - nf * _LN2_HI - nf * _LN2_LO
    p = 1.0 + r * (1.0 + r * (0.5 + r * (0.16666667 + r * (0.041666668 + r * 0.008333334))))
    scale = plsc.bitcast(jnp.left_shift(n + 127, 23), jnp.float32)
    return p * scale


def _inv(d):
    r = 1.0 / d
    return r * (2.0 - d * r)


def _col48():
    i = lax.broadcasted_iota(jnp.int32, (1, HPAD), 1)
    return jnp.where(i == 48, 1.0, 0.0).astype(jnp.float32)


def _dense_tail(h_pre, a_s, a_d, h_out, as_out, ad_out, ms_out):
    h = h_pre + _col48()
    h_out[...] = h
    al_s = jnp.sum(h * a_s, axis=1)
    al_d = jnp.sum(h * a_d, axis=1)
    as_out[...] = al_s[None, :]
    ad_out[...] = al_d[None, :]
    ms_out[...] = jnp.full((1, 128), jnp.max(al_s), dtype=jnp.float32)


def _tc_first_body(x_ref, w_ref, as_ref, ad_ref, h_out, as_out, ad_out, ms_out):
    h_pre = jnp.dot(x_ref[...], w_ref[...], preferred_element_type=jnp.float32,
                    precision=jax.lax.Precision.HIGHEST)
    _dense_tail(h_pre, as_ref[...], ad_ref[...], h_out, as_out, ad_out, ms_out)


def _tc_mid_body(p0_ref, p1_ref, b_ref, w_ref, as_ref, ad_ref,
                 h_out, as_out, ad_out, ms_out):
    P = p0_ref[...] + p1_ref[...]
    mask48 = _col48()
    den = jnp.sum(P * mask48, axis=1, keepdims=True)
    y = P * _inv(den + 1e-16) + b_ref[...]
    y = jnp.where(y > 0, y, jnp.exp(y) - 1.0)
    h_pre = jnp.dot(y, w_ref[...], preferred_element_type=jnp.float32,
                    precision=jax.lax.Precision.HIGHEST)
    _dense_tail(h_pre, as_ref[...], ad_ref[...], h_out, as_out, ad_out, ms_out)


def _tc_final_body(p0_ref, p1_ref, b_ref, y_out):
    P = p0_ref[...] + p1_ref[...]
    mask48 = _col48()
    den = jnp.sum(P * mask48, axis=1, keepdims=True)
    y = P * _inv(den + 1e-16) + b_ref[...]
    y_out[...] = jnp.where(y > 0, y, jnp.exp(y) - 1.0)


_DENSE_OUT = (
    jax.ShapeDtypeStruct((NPAD, HPAD), jnp.float32),
    jax.ShapeDtypeStruct((1, NPAD), jnp.float32),
    jax.ShapeDtypeStruct((1, NPAD), jnp.float32),
    jax.ShapeDtypeStruct((1, 128), jnp.float32),
)

_tc_first = pl.pallas_call(_tc_first_body, out_shape=_DENSE_OUT)
_tc_mid = pl.pallas_call(_tc_mid_body, out_shape=_DENSE_OUT)
_tc_final = pl.pallas_call(
    _tc_final_body, out_shape=jax.ShapeDtypeStruct((NPAD, HPAD), jnp.float32))


def _sc_edge_body(src_hbm, dst_hbm, h_hbm, as_hbm, ad_hbm, ms_hbm, out_hbm,
                  asv, adv, msv, srcv, dstv, rows, rows_b, acc, sem, sem_b):
    c = lax.axis_index("c")
    s = lax.axis_index("s")
    wid = c * 16 + s

    pltpu.sync_copy(as_hbm, asv)
    pltpu.sync_copy(ad_hbm, adv)
    pltpu.sync_copy(ms_hbm, msv)
    pltpu.sync_copy(src_hbm.at[wid], srcv)
    pltpu.sync_copy(dst_hbm.at[wid], dstv)

    zero16 = jnp.zeros((16,), jnp.float32)

    def _zrow(i, _):
        for q in range(4):
            rows[i, pl.ds(q * 16, 16)] = zero16
        return 0

    lax.fori_loop(0, K, _zrow, 0)
    for kk in range(ROWS_PER_TILE // K):
        pltpu.sync_copy(rows, acc.at[pl.ds(s * ROWS_PER_TILE + kk * K, K), :])
    plsc.subcore_barrier()

    msvec = msv[...]

    def _scale_and_scatter(g, buf):
        for j in range(K // 16):
            idxs = srcv[g, pl.ds(j * 16, 16)]
            idxd = dstv[g, pl.ds(j * 16, 16)]
            a_s_g = plsc.load_gather(asv, [idxs])
            a_d_g = plsc.load_gather(adv, [idxd])
            e = a_s_g + a_d_g
            e = jnp.maximum(e, NEG * e)
            cb = msvec + a_d_g
            cb = jnp.maximum(cb, NEG * cb)
            ex = _soft_exp(e - cb)
            for t in range(16):
                spl = jnp.full((16,), ex[t], jnp.float32)
                r = j * 16 + t
                for q in range(4):
                    buf[r, pl.ds(q * 16, 16)] = buf[r, pl.ds(q * 16, 16)] * spl
        pltpu.sync_copy(buf, acc.at[dstv.at[g]], add=True)

    pltpu.async_copy(h_hbm.at[srcv.at[0]], rows, sem)

    def _pair(p, _):
        ga = 2 * p
        gb = 2 * p + 1
        pltpu.async_copy(h_hbm.at[srcv.at[gb]], rows_b, sem_b)
        pltpu.make_async_copy(h_hbm.at[srcv.at[ga]], rows, sem).wait()
        _scale_and_scatter(ga, rows)

        @pl.when(ga + 2 < NCHUNK)
        def _():
            pltpu.async_copy(h_hbm.at[srcv.at[ga + 2]], rows, sem)

        pltpu.make_async_copy(h_hbm.at[srcv.at[gb]], rows_b, sem_b).wait()
        _scale_and_scatter(gb, rows_b)
        return 0

    lax.fori_loop(0, NCHUNK // 2, _pair, 0)

    plsc.subcore_barrier()
    pltpu.sync_copy(acc.at[pl.ds(s * ROWS_PER_TILE, ROWS_PER_TILE), :],
                    out_hbm.at[c, pl.ds(s * ROWS_PER_TILE, ROWS_PER_TILE), :])


_sc_edge = functools.partial(
    pl.kernel,
    out_type=jax.ShapeDtypeStruct((2, NPAD, HPAD), jnp.float32),
    mesh=plsc.VectorSubcoreMesh(core_axis_name="c", subcore_axis_name="s"),
    compiler_params=pltpu.CompilerParams(needs_layout_passes=False,
                                         use_tc_tiling_on_sc=False),
    scratch_types=[
        pltpu.VMEM((NPAD,), jnp.float32),
        pltpu.VMEM((NPAD,), jnp.float32),
        pltpu.VMEM((16,), jnp.float32),
        pltpu.VMEM((NCHUNK, K), jnp.int32),
        pltpu.VMEM((NCHUNK, K), jnp.int32),
        pltpu.VMEM((K, HPAD), jnp.float32),
        pltpu.VMEM((K, HPAD), jnp.float32),
        pltpu.VMEM_SHARED((NPAD, HPAD), jnp.float32),
        pltpu.SemaphoreType.DMA,
        pltpu.SemaphoreType.DMA,
    ],
)(_sc_edge_body)


def kernel(x, edge_index, W0, a_src0, a_dst0, b0, W1, a_src1, a_dst1, b1,
           W2, a_src2, a_dst2, b2, W3, a_src3, a_dst3, b3,
           W4, a_src4, a_dst4, b4):
    xp = jnp.pad(x, ((0, NPAD - N), (0, 0)))
    loops = jnp.arange(N, dtype=edge_index.dtype)
    npad_e = EPAD - N - edge_index.shape[1]
    src = jnp.concatenate(
        [edge_index[0], loops, jnp.full((npad_e,), N, edge_index.dtype)])
    dst = jnp.concatenate(
        [edge_index[1], loops, jnp.full((npad_e,), N, edge_index.dtype)])
    src3 = src.reshape(NTILES, NCHUNK, K)
    dst3 = dst.reshape(NTILES, NCHUNK, K)

    Ws = [W0, W1, W2, W3, W4]
    avs = [(a_src0, a_dst0), (a_src1, a_dst1), (a_src2, a_dst2),
           (a_src3, a_dst3), (a_src4, a_dst4)]
    bs = [b0, b1, b2, b3, b4]
    Wp = [jnp.pad(W, ((0, (D_IN if i == 0 else HPAD) - W.shape[0]),
                      (0, HPAD - HID))) for i, W in enumerate(Ws)]
    ap = [(jnp.pad(a_s, (0, HPAD - HID))[None, :],
           jnp.pad(a_d, (0, HPAD - HID))[None, :]) for a_s, a_d in avs]
    bp = [jnp.pad(b, (0, HPAD - HID))[None, :] for b in bs]

    h, al_s, al_d, ms = _tc_first(xp, Wp[0], ap[0][0], ap[0][1])
    for i in range(DEPTH):
        part = _sc_edge(src3, dst3, h,
                        al_s.reshape(NPAD), al_d.reshape(NPAD),
                        ms.reshape(128)[:16])
        if i < DEPTH - 1:
            h, al_s, al_d, ms = _tc_mid(part[0], part[1], bp[i], Wp[i + 1],
                                        ap[i + 1][0], ap[i + 1][1])
        else:
            y = _tc_final(part[0], part[1], bp[i])
    return y[:N, :HID]


# speedup vs baseline: 1.1577x; 1.1577x over previous
import functools

import jax
import jax.numpy as jnp
from jax import lax
from jax.experimental import pallas as pl
from jax.experimental.pallas import tpu as pltpu
from jax.experimental.pallas import tpu_sc as plsc

N = 10000
D_IN = 128
HID = 48
NEG = 0.2
DEPTH = 5

NPAD = 10240
HPAD = 64
NTILES = 32
EP_TILE = 10368
EPAD = NTILES * EP_TILE
K = 128
NCHUNK = EP_TILE // K
ROWS_PER_TILE = NPAD // 16


_LOG2E = 1.4426950408889634
_LN2_HI = 0.693145751953125
_LN2_LO = 1.42860677e-06


def _soft_exp(x):
    x = jnp.maximum(x, -80.0)
    n = (x * _LOG2E - 0.5).astype(jnp.int32)
    nf = n.astype(jnp.float32)
    r = x - nf * _LN2_HI - nf * _LN2_LO
    p = 1.0 + r * (1.0 + r * (0.5 + r * (0.16666667 + r * (0.041666668 + r * 0.008333334))))
    scale = plsc.bitcast(jnp.left_shift(n + 127, 23), jnp.float32)
    return p * scale


def _inv(d):
    r = 1.0 / d
    return r * (2.0 - d * r)


def _col48():
    i = lax.broadcasted_iota(jnp.int32, (1, HPAD), 1)
    return jnp.where(i == 48, 1.0, 0.0).astype(jnp.float32)


def _dense_tail(h_pre, a_s, a_d, h_out, as_out, ad_out, ms_out):
    h = h_pre + _col48()
    h_out[...] = h
    al_s = jnp.sum(h * a_s, axis=1)
    al_d = jnp.sum(h * a_d, axis=1)
    as_out[...] = al_s[None, :]
    ad_out[...] = al_d[None, :]
    ms_out[...] = jnp.full((1, 128), jnp.max(al_s), dtype=jnp.float32)


def _tc_first_body(x_ref, w_ref, as_ref, ad_ref, h_out, as_out, ad_out, ms_out):
    h_pre = jnp.dot(x_ref[...], w_ref[...], preferred_element_type=jnp.float32,
                    precision=jax.lax.Precision.HIGHEST)
    _dense_tail(h_pre, as_ref[...], ad_ref[...], h_out, as_out, ad_out, ms_out)


def _tc_mid_body(p0_ref, p1_ref, b_ref, w_ref, as_ref, ad_ref,
                 h_out, as_out, ad_out, ms_out):
    P = p0_ref[...] + p1_ref[...]
    mask48 = _col48()
    den = jnp.sum(P * mask48, axis=1, keepdims=True)
    y = P * _inv(den + 1e-16) + b_ref[...]
    y = jnp.where(y > 0, y, jnp.exp(y) - 1.0)
    h_pre = jnp.dot(y, w_ref[...], preferred_element_type=jnp.float32,
                    precision=jax.lax.Precision.HIGHEST)
    _dense_tail(h_pre, as_ref[...], ad_ref[...], h_out, as_out, ad_out, ms_out)


def _tc_final_body(p0_ref, p1_ref, b_ref, y_out):
    P = p0_ref[...] + p1_ref[...]
    mask48 = _col48()
    den = jnp.sum(P * mask48, axis=1, keepdims=True)
    y = P * _inv(den + 1e-16) + b_ref[...]
    y_out[...] = jnp.where(y > 0, y, jnp.exp(y) - 1.0)


_DENSE_OUT = (
    jax.ShapeDtypeStruct((NPAD, HPAD), jnp.float32),
    jax.ShapeDtypeStruct((1, NPAD), jnp.float32),
    jax.ShapeDtypeStruct((1, NPAD), jnp.float32),
    jax.ShapeDtypeStruct((1, 128), jnp.float32),
)

_tc_first = pl.pallas_call(_tc_first_body, out_shape=_DENSE_OUT)
_tc_mid = pl.pallas_call(_tc_mid_body, out_shape=_DENSE_OUT)
_tc_final = pl.pallas_call(
    _tc_final_body, out_shape=jax.ShapeDtypeStruct((NPAD, HPAD), jnp.float32))


def _sc_edge_body(src_hbm, dst_hbm, h_hbm, as_hbm, ad_hbm, ms_hbm, out_hbm,
                  asv, adv, msv, srcv, dstv, rows, acc, sem):
    c = lax.axis_index("c")
    s = lax.axis_index("s")
    wid = c * 16 + s

    pltpu.sync_copy(as_hbm, asv)
    pltpu.sync_copy(ad_hbm, adv)
    pltpu.sync_copy(ms_hbm, msv)
    pltpu.sync_copy(src_hbm.at[wid], srcv)
    pltpu.sync_copy(dst_hbm.at[wid], dstv)

    zero16 = jnp.zeros((16,), jnp.float32)

    def _zrow(i, _):
        for q in range(4):
            rows[i, pl.ds(q * 16, 16)] = zero16
        return 0

    lax.fori_loop(0, K, _zrow, 0)
    for kk in range(ROWS_PER_TILE // K):
        pltpu.sync_copy(rows, acc.at[pl.ds(s * ROWS_PER_TILE + kk * K, K), :])
    plsc.subcore_barrier()

    msvec = msv[...]

    def _scale_and_scatter(g, buf):
        for j in range(K // 16):
            idxs = srcv[g, pl.ds(j * 16, 16)]
            idxd = dstv[g, pl.ds(j * 16, 16)]
            a_s_g = plsc.load_gather(asv, [idxs])
            a_d_g = plsc.load_gather(adv, [idxd])
            e = a_s_g + a_d_g
            e = jnp.maximum(e, NEG * e)
            cb = msvec + a_d_g
            cb = jnp.maximum(cb, NEG * cb)
            ex = _soft_exp(e - cb)
            for t in range(16):
                spl = jnp.full((16,), ex[t], jnp.float32)
                r = j * 16 + t
                for q in range(4):
                    buf[r, pl.ds(q * 16, 16)] = buf[r, pl.ds(q * 16, 16)] * spl
        pltpu.sync_copy(buf, acc.at[dstv.at[g]], add=True)

    def _chunk(g, _):
        pltpu.sync_copy(h_hbm.at[srcv.at[g]], rows)
        _scale_and_scatter(g, rows)
        return 0

    lax.fori_loop(0, NCHUNK, _chunk, 0)

    plsc.subcore_barrier()
    pltpu.sync_copy(acc.at[pl.ds(s * ROWS_PER_TILE, ROWS_PER_TILE), :],
                    out_hbm.at[c, pl.ds(s * ROWS_PER_TILE, ROWS_PER_TILE), :])


_sc_edge = functools.partial(
    pl.kernel,
    out_type=jax.ShapeDtypeStruct((2, NPAD, HPAD), jnp.float32),
    mesh=plsc.VectorSubcoreMesh(core_axis_name="c", subcore_axis_name="s"),
    compiler_params=pltpu.CompilerParams(needs_layout_passes=False,
                                         use_tc_tiling_on_sc=False),
    scratch_types=[
        pltpu.VMEM((NPAD,), jnp.float32),
        pltpu.VMEM((NPAD,), jnp.float32),
        pltpu.VMEM((16,), jnp.float32),
        pltpu.VMEM((NCHUNK, K), jnp.int32),
        pltpu.VMEM((NCHUNK, K), jnp.int32),
        pltpu.VMEM((K, HPAD), jnp.float32),
        pltpu.VMEM_SHARED((NPAD, HPAD), jnp.float32),
        pltpu.SemaphoreType.DMA,
    ],
)(_sc_edge_body)


def kernel(x, edge_index, W0, a_src0, a_dst0, b0, W1, a_src1, a_dst1, b1,
           W2, a_src2, a_dst2, b2, W3, a_src3, a_dst3, b3,
           W4, a_src4, a_dst4, b4):
    xp = jnp.pad(x, ((0, NPAD - N), (0, 0)))
    loops = jnp.arange(N, dtype=edge_index.dtype)
    npad_e = EPAD - N - edge_index.shape[1]
    src = jnp.concatenate(
        [edge_index[0], loops, jnp.full((npad_e,), N, edge_index.dtype)])
    dst = jnp.concatenate(
        [edge_index[1], loops, jnp.full((npad_e,), N, edge_index.dtype)])
    src3 = src.reshape(NTILES, NCHUNK, K)
    dst3 = dst.reshape(NTILES, NCHUNK, K)

    Ws = [W0, W1, W2, W3, W4]
    avs = [(a_src0, a_dst0), (a_src1, a_dst1), (a_src2, a_dst2),
           (a_src3, a_dst3), (a_src4, a_dst4)]
    bs = [b0, b1, b2, b3, b4]
    Wp = [jnp.pad(W, ((0, (D_IN if i == 0 else HPAD) - W.shape[0]),
                      (0, HPAD - HID))) for i, W in enumerate(Ws)]
    ap = [(jnp.pad(a_s, (0, HPAD - HID))[None, :],
           jnp.pad(a_d, (0, HPAD - HID))[None, :]) for a_s, a_d in avs]
    bp = [jnp.pad(b, (0, HPAD - HID))[None, :] for b in bs]

    h, al_s, al_d, ms = _tc_first(xp, Wp[0], ap[0][0], ap[0][1])
    for i in range(DEPTH):
        part = _sc_edge(src3, dst3, h,
                        al_s.reshape(NPAD), al_d.reshape(NPAD),
                        ms.reshape(128)[:16])
        if i < DEPTH - 1:
            h, al_s, al_d, ms = _tc_mid(part[0], part[1], bp[i], Wp[i + 1],
                                        ap[i + 1][0], ap[i + 1][1])
        else:
            y = _tc_final(part[0], part[1], bp[i])
    return y[:N, :HID]
